# baseline (speedup 1.0000x reference)
import jax
import jax.numpy as jnp
from jax.experimental import pallas as pl

_N = 50000
_G = 16
_C = 8
_NUM_FEAT = 123
_NUM_CFG_FEAT = 18
_NUM_OPS = 125
_DIM = 128

_BLOCK = 2000


def _fused_kernel(code_ref, tidx_ref, opf_ref, cfg_ref, opw_ref, cfgw_ref,
                  emb_ref, wopT_ref, bop_ref, wcfgT_ref, bcfg_ref, out_ref):
    code = code_ref[:, :]
    oh_op = (code == jax.lax.broadcasted_iota(jnp.int32, (1, _NUM_OPS), 1)
             ).astype(jnp.float32)
    row = jnp.dot(oh_op, emb_ref[:, :], preferred_element_type=jnp.float32)
    sq = jnp.sum(row * row, axis=1, keepdims=True)
    scale = jnp.where(sq > 1.0, jax.lax.rsqrt(sq), 1.0)
    op_emb = opw_ref[0, 0] * (row * scale)

    x = (jnp.dot(opf_ref[:, :], wopT_ref[:, :],
                 preferred_element_type=jnp.float32)
         + bop_ref[0, :][None, :] + op_emb)

    scaled_cfg = cfg_ref[:, :, :] * cfgw_ref[0, :][None, None, :]
    cfg_flat = scaled_cfg.reshape(_G * _C, _NUM_CFG_FEAT)
    cfg_all = (jnp.dot(cfg_flat, wcfgT_ref[:, :],
                       preferred_element_type=jnp.float32)
               + bcfg_ref[0, :][None, :])

    oh2 = (tidx_ref[:, :] ==
           jax.lax.broadcasted_iota(jnp.int32, (1, _G * _C), 1)
           ).astype(jnp.float32)
    cfg_pn = jnp.dot(oh2, cfg_all,
                     preferred_element_type=jnp.float32)
    out_ref[:, :, :] = x[:, None, :] + cfg_pn.reshape(_BLOCK, _C, _DIM)


def kernel(op_code, op_feats, config_feats, batch_idx, op_weights,
           config_weights, emb_table, W_op, b_op, W_cfg, b_cfg):
    n = op_feats.shape[0]
    code2 = op_code.reshape(n, 1).astype(jnp.int32)
    tidx = (batch_idx.astype(jnp.int32)[:, None] * _C
            + jnp.arange(_C, dtype=jnp.int32)[None, :]).reshape(n * _C, 1)
    cfgw2 = config_weights.reshape(1, _NUM_CFG_FEAT)
    bop2 = b_op.reshape(1, _DIM)
    bcfg2 = b_cfg.reshape(1, _DIM)
    wopT = W_op.T
    wcfgT = W_cfg.T

    nb = n // _BLOCK
    grid = (nb,)

    def whole(shape):
        return pl.BlockSpec(shape, lambda i: (0,) * len(shape))

    out = pl.pallas_call(
        _fused_kernel,
        grid=grid,
        in_specs=[
            pl.BlockSpec((_BLOCK, 1), lambda i: (i, 0)),
            pl.BlockSpec((_BLOCK * _C, 1), lambda i: (i, 0)),
            pl.BlockSpec((_BLOCK, _NUM_FEAT), lambda i: (i, 0)),
            whole((_G, _C, _NUM_CFG_FEAT)),
            whole((1, 1)),
            whole((1, _NUM_CFG_FEAT)),
            whole((_NUM_OPS, _DIM)),
            whole((_NUM_FEAT, _DIM)),
            whole((1, _DIM)),
            whole((_NUM_CFG_FEAT, _DIM)),
            whole((1, _DIM)),
        ],
        out_specs=pl.BlockSpec((_BLOCK, _C, _DIM), lambda i: (i, 0, 0)),
        out_shape=jax.ShapeDtypeStruct((n, _C, _DIM), jnp.float32),
    )(code2, tidx, op_feats, config_feats, op_weights, cfgw2,
      emb_table, wopT, bop2, wcfgT, bcfg2)
    return out


# speedup vs baseline: 2.1508x; 2.1508x over previous
import jax
import jax.numpy as jnp
from jax.experimental import pallas as pl

_N = 50000
_G = 16
_C = 8
_NUM_FEAT = 123
_NUM_CFG_FEAT = 18
_NUM_OPS = 125
_DIM = 128

_BLOCK = 2000


def _fused_kernel(code_ref, bidx_ref, opf_ref, cfg_ref, opw_ref, cfgw_ref,
                  emb_ref, wopT_ref, bop_ref, wcfgT_ref, bcfg_ref, out_ref):
    code = code_ref[:, :]
    oh_op = (code == jax.lax.broadcasted_iota(jnp.int32, (1, _NUM_OPS), 1)
             ).astype(jnp.float32)
    row = jnp.dot(oh_op, emb_ref[:, :], preferred_element_type=jnp.float32)
    sq = jnp.sum(row * row, axis=1, keepdims=True)
    scale = jnp.where(sq > 1.0, jax.lax.rsqrt(sq), 1.0)
    op_emb = opw_ref[0, 0] * (row * scale)

    x = (jnp.dot(opf_ref[:, :], wopT_ref[:, :],
                 preferred_element_type=jnp.float32)
         + bop_ref[0, :][None, :] + op_emb)

    oh_g = (bidx_ref[:, :] == jax.lax.broadcasted_iota(jnp.int32, (1, _G), 1)
            ).astype(jnp.float32)
    scaled_cfg = cfg_ref[:, :, :] * cfgw_ref[0, :][None, None, :]
    cfg_flat = scaled_cfg.reshape(_G * _C, _NUM_CFG_FEAT)
    cfg_all = (jnp.dot(cfg_flat, wcfgT_ref[:, :],
                       preferred_element_type=jnp.float32)
               + bcfg_ref[0, :][None, :]).reshape(_G, _C, _DIM)
    cfg_pn = jax.lax.dot_general(
        oh_g, cfg_all, (((1,), (0,)), ((), ())),
        preferred_element_type=jnp.float32)
    out_ref[:, :, :] = x[:, None, :] + cfg_pn


def kernel(op_code, op_feats, config_feats, batch_idx, op_weights,
           config_weights, emb_table, W_op, b_op, W_cfg, b_cfg):
    n = op_feats.shape[0]
    code2 = op_code.reshape(n, 1).astype(jnp.int32)
    bidx2 = batch_idx.reshape(n, 1).astype(jnp.int32)
    cfgw2 = config_weights.reshape(1, _NUM_CFG_FEAT)
    bop2 = b_op.reshape(1, _DIM)
    bcfg2 = b_cfg.reshape(1, _DIM)
    wopT = W_op.T
    wcfgT = W_cfg.T

    nb = n // _BLOCK
    grid = (nb,)

    def row_block(shape_tail):
        return pl.BlockSpec((_BLOCK,) + shape_tail,
                            lambda i: (i,) + (0,) * len(shape_tail))

    def whole(shape):
        return pl.BlockSpec(shape, lambda i: (0,) * len(shape))

    out = pl.pallas_call(
        _fused_kernel,
        grid=grid,
        in_specs=[
            row_block((1,)),
            row_block((1,)),
            row_block((_NUM_FEAT,)),
            whole((_G, _C, _NUM_CFG_FEAT)),
            whole((1, 1)),
            whole((1, _NUM_CFG_FEAT)),
            whole((_NUM_OPS, _DIM)),
            whole((_NUM_FEAT, _DIM)),
            whole((1, _DIM)),
            whole((_NUM_CFG_FEAT, _DIM)),
            whole((1, _DIM)),
        ],
        out_specs=pl.BlockSpec((_BLOCK, _C, _DIM), lambda i: (i, 0, 0)),
        out_shape=jax.ShapeDtypeStruct((n, _C, _DIM), jnp.float32),
    )(code2, bidx2, op_feats, config_feats, op_weights, cfgw2,
      emb_table, wopT, bop2, wcfgT, bcfg2)
    return out
